# Initial kernel scaffold; baseline (speedup 1.0000x reference)
import jax
import jax.numpy as jnp
from jax.experimental import pallas as pl


def kernel(token_ids, weight):
    raise NotImplementedError("write your pallas kernel here")


# speedup vs baseline: 1.6850x; 1.6850x over previous
import functools

import jax
import jax.numpy as jnp
from jax import lax
from jax.experimental import pallas as pl
from jax.experimental.pallas import tpu as pltpu
from jax.experimental.pallas import tpu_sc as plsc

_INFO = plsc.get_sparse_core_info()
_NC = _INFO.num_cores
_NS = _INFO.num_subcores
_NW = _NC * _NS
_CW = 128


def _embed_lookup(B, D, n_chunks):
    mesh = plsc.VectorSubcoreMesh(core_axis_name="c", subcore_axis_name="s")

    @functools.partial(
        pl.kernel,
        mesh=mesh,
        out_type=jax.ShapeDtypeStruct((B, D), jnp.float32),
        scratch_types=[
            pltpu.VMEM((n_chunks, _CW), jnp.int32),
            pltpu.VMEM((_CW, D), jnp.float32),
            pltpu.SemaphoreType.DMA,
        ],
        compiler_params=pltpu.CompilerParams(use_tc_tiling_on_sc=False),
    )
    def k(idx_hbm, table_hbm, out_hbm, idx_v, rows_v, sem):
        wid = lax.axis_index("s") * _NC + lax.axis_index("c")
        pltpu.sync_copy(idx_hbm.at[wid], idx_v)
        base = wid * (n_chunks * _CW)

        def body(j, carry):
            pltpu.async_copy(table_hbm.at[idx_v.at[j]], rows_v, sem).wait()
            pltpu.sync_copy(rows_v, out_hbm.at[pl.ds(base + j * _CW, _CW)])
            return carry

        lax.fori_loop(0, n_chunks, body, 0)

    return k


def kernel(token_ids, weight):
    B0, B1 = token_ids.shape
    V, D = weight.shape
    B = B0 * B1
    n_chunks = B // (_NW * _CW)
    idx = token_ids.reshape(_NW, n_chunks, _CW).astype(jnp.int32)
    out = _embed_lookup(B, D, n_chunks)(idx, weight)
    return out.reshape(B0, B1, D)


# speedup vs baseline: 1.8689x; 1.1092x over previous
import functools

import jax
import jax.numpy as jnp
from jax import lax
from jax.experimental import pallas as pl
from jax.experimental.pallas import tpu as pltpu
from jax.experimental.pallas import tpu_sc as plsc

_INFO = plsc.get_sparse_core_info()
_NC = _INFO.num_cores
_NS = _INFO.num_subcores
_NW = _NC * _NS
_CW = 128


_K = 4


def _embed_lookup(B, D, n_chunks):
    mesh = plsc.VectorSubcoreMesh(core_axis_name="c", subcore_axis_name="s")
    n_groups = n_chunks // _K
    assert n_chunks % _K == 0 and n_groups % 2 == 0

    @functools.partial(
        pl.kernel,
        mesh=mesh,
        out_type=jax.ShapeDtypeStruct((B, D), jnp.float32),
        scratch_types=[
            pltpu.VMEM((n_chunks, _CW), jnp.int32),
            pltpu.VMEM((2, _K, _CW, D), jnp.float32),
            pltpu.SemaphoreType.DMA,
            pltpu.SemaphoreType.DMA,
            pltpu.SemaphoreType.DMA,
            pltpu.SemaphoreType.DMA,
        ],
        compiler_params=pltpu.CompilerParams(use_tc_tiling_on_sc=False),
    )
    def k(idx_hbm, table_hbm, out_hbm, idx_v, rows_v, gsem_a, gsem_b, ssem_a, ssem_b):
        wid = lax.axis_index("s") * _NC + lax.axis_index("c")
        pltpu.sync_copy(idx_hbm.at[wid], idx_v)
        base = wid * (n_chunks * _CW)

        def gathers(half, gsem, g):
            for b in range(_K):
                pltpu.async_copy(
                    table_hbm.at[idx_v.at[g * _K + b]], rows_v.at[half, b], gsem)

        def wait_gathers(half, gsem, g):
            for b in range(_K):
                pltpu.make_async_copy(
                    table_hbm.at[idx_v.at[g * _K + b]], rows_v.at[half, b], gsem
                ).wait()

        def stores(half, ssem, g):
            for b in range(_K):
                pltpu.async_copy(
                    rows_v.at[half, b],
                    out_hbm.at[pl.ds(base + (g * _K + b) * _CW, _CW)], ssem)

        def wait_stores(half, ssem, g):
            for b in range(_K):
                pltpu.make_async_copy(
                    rows_v.at[half, b],
                    out_hbm.at[pl.ds(base + (g * _K + b) * _CW, _CW)], ssem
                ).wait()

        gathers(0, gsem_a, 0)

        def body(gg, carry):
            g0 = 2 * gg
            g1 = g0 + 1
            wait_gathers(0, gsem_a, g0)

            @pl.when(gg > 0)
            def _():
                wait_stores(1, ssem_b, g1 - 2)

            gathers(1, gsem_b, g1)
            stores(0, ssem_a, g0)
            wait_gathers(1, gsem_b, g1)
            wait_stores(0, ssem_a, g0)

            @pl.when(g0 + 2 < n_groups)
            def _():
                gathers(0, gsem_a, g0 + 2)

            stores(1, ssem_b, g1)
            return carry

        lax.fori_loop(0, n_groups // 2, body, 0)
        wait_stores(1, ssem_b, n_groups - 1)

    return k


def kernel(token_ids, weight):
    B0, B1 = token_ids.shape
    V, D = weight.shape
    B = B0 * B1
    n_chunks = B // (_NW * _CW)
    idx = token_ids.reshape(_NW, n_chunks, _CW).astype(jnp.int32)
    out = _embed_lookup(B, D, n_chunks)(idx, weight)
    return out.reshape(B0, B1, D)
